# baseline (speedup 1.0000x reference)
import functools

import jax
import jax.numpy as jnp
from jax.experimental import pallas as pl
from jax.experimental.pallas import tpu as pltpu


def _persample_kernel(x1_ref, x2_ref, out_ref, *, n_total):
    C = x1_ref.shape[1]
    S = x1_ref.shape[2]
    split = S // 3
    sizes = (split, split, S - 2 * split)
    ones_row = jnp.ones((1, S), jnp.float32)

    def heat_and_rank(x):
        heat = jnp.sum(x, axis=0, keepdims=True) * (1.0 / C)
        heat_col = jnp.transpose(heat)
        beats = (heat_col > heat).astype(jnp.float32)
        rank0 = jax.lax.dot_general(
            ones_row, beats, (((1,), (0,)), ((), ()))
        )
        return heat, rank0

    h1, r1 = heat_and_rank(x1_ref[0])
    h2, r2 = heat_and_rank(x2_ref[0])
    H = jnp.concatenate([h1, h2], axis=0)
    R = jnp.concatenate([r1, r2], axis=0)

    inf = jnp.float32(jnp.inf)
    t1 = jnp.min(jnp.where(R <= float(split - 1), H, inf), axis=1, keepdims=True)
    t2 = jnp.min(
        jnp.where(R <= float(2 * split - 1), H, inf), axis=1, keepdims=True
    )
    ngt1 = jnp.sum((H > t1).astype(jnp.float32), axis=1, keepdims=True)
    ngt2 = jnp.sum((H > t2).astype(jnp.float32), axis=1, keepdims=True)
    eq1 = (H == t1).astype(jnp.float32)
    eq2 = (H == t2).astype(jnp.float32)

    EQ = jnp.concatenate([eq1, eq2], axis=0)
    lane = jax.lax.broadcasted_iota(jnp.int32, (4, S), 1)
    scan = EQ
    d = 1
    while d < S:
        scan = scan + jnp.where(lane >= d, pltpu.roll(scan, d, 1), 0.0)
        d *= 2
    PE = scan - EQ
    pe1, pe2 = PE[0:2], PE[2:4]

    m0 = jnp.where(
        (H > t1) | ((eq1 > 0.0) & (ngt1 + pe1 < float(split))), 1.0, 0.0
    )
    m01 = jnp.where(
        (H > t2) | ((eq2 > 0.0) & (ngt2 + pe2 < float(2 * split))), 1.0, 0.0
    )

    rows = jnp.concatenate(
        [m0, m01, jnp.ones((1, S), jnp.float32), jnp.zeros((3, S), jnp.float32)],
        axis=0,
    )
    rhs = jnp.transpose(rows)

    def block_means(x, c0, c01):
        sums = jax.lax.dot_general(
            x, rhs, (((1,), (0,)), ((), ())),
            precision=jax.lax.Precision.HIGHEST,
        )
        s0 = sums[:, c0 : c0 + 1]
        s01 = sums[:, c01 : c01 + 1]
        tot = sums[:, 4:5]
        return (
            s0 * (1.0 / sizes[0]),
            (s01 - s0) * (1.0 / sizes[1]),
            (tot - s01) * (1.0 / sizes[2]),
        )

    b1 = block_means(x1_ref[0], 0, 2)
    b2 = block_means(x2_ref[0], 1, 3)
    nsq1 = sum(jnp.sum(m * m) for m in b1)
    nsq2 = sum(jnp.sum(m * m) for m in b2)
    inv1 = 1.0 / jnp.maximum(jnp.sqrt(nsq1), 1e-12)
    inv2 = 1.0 / jnp.maximum(jnp.sqrt(nsq2), 1e-12)
    dsq = sum(jnp.sum((a * inv1 - b * inv2) ** 2) for a, b in zip(b1, b2))
    out_ref[0, 0, :] = jnp.full((128,), dsq * (1.0 / n_total), jnp.float32)


def kernel(image_features1, image_features2, logit_scale, weights, blocks):
    B, C, H, W = image_features1.shape
    S = H * W
    x1 = image_features1.reshape(B, C, S)
    x2 = image_features2.reshape(B, C, S)
    body = functools.partial(_persample_kernel, n_total=B * 3 * C)
    out = pl.pallas_call(
        body,
        grid=(B,),
        in_specs=[
            pl.BlockSpec((1, C, S), lambda b: (b, 0, 0)),
            pl.BlockSpec((1, C, S), lambda b: (b, 0, 0)),
        ],
        out_specs=pl.BlockSpec((1, 1, 128), lambda b: (b, 0, 0)),
        out_shape=jax.ShapeDtypeStruct((B, 1, 128), jnp.float32),
        compiler_params=pltpu.CompilerParams(
            dimension_semantics=("parallel",)
        ),
    )(x1, x2)
    return jnp.sum(out[:, 0, 0])
